# baseline (speedup 1.0000x reference)
import functools

import jax
import jax.numpy as jnp
from jax import lax
from jax.experimental import pallas as pl
from jax.experimental.pallas import tpu as pltpu
from jax.experimental.pallas import tpu_sc as plsc

_BATCH = 4096
_MAX_LEN = 200
_EMBED = 32
_VOCAB = 1000000
_B = _BATCH * _MAX_LEN
_NC = 2
_NS = 16
_NW = _NC * _NS
_NCOLT = 7813
_KMAX = 245

_BPW = _B // _NW
_CHUNK = 1600
_NCHUNK = _BPW // _CHUNK


def _transpose_block(src, dst, ncg, i32):
    for cg in range(ncg):
        for e in range(_EMBED):
            v = src[e, pl.ds(cg * 16, 16)]
            plsc.store_scatter(dst, [i32 + (cg * 512 + e)], v)


@jax.jit
def _table_to_rowmajor(table_t, tail_flat):
    mesh = plsc.VectorSubcoreMesh(core_axis_name="c", subcore_axis_name="s")

    @functools.partial(
        pl.kernel,
        mesh=mesh,
        out_type=jax.ShapeDtypeStruct((_VOCAB * _EMBED,), jnp.float32),
        scratch_types=[
            pltpu.VMEM((_EMBED, 128), jnp.float32),
            pltpu.VMEM((_EMBED, 128), jnp.float32),
            pltpu.VMEM((4096,), jnp.float32),
            pltpu.VMEM((4096,), jnp.float32),
            pltpu.VMEM((2048,), jnp.float32),
            pltpu.SemaphoreType.DMA,
            pltpu.SemaphoreType.DMA,
            pltpu.SemaphoreType.DMA,
            pltpu.SemaphoreType.DMA,
        ],
        compiler_params=pltpu.CompilerParams(
            use_tc_tiling_on_sc=True, needs_layout_passes=False),
    )
    def k(tt_hbm, tail_hbm, tlin_hbm, src_v0, src_v1, dst_v0, dst_v1,
          tail_v, isem0, isem1, osem0, osem1):
        src_v = (src_v0, src_v1)
        dst_v = (dst_v0, dst_v1)
        isem = (isem0, isem1)
        osem = (osem0, osem1)
        wid = lax.axis_index("s") * _NC + lax.axis_index("c")
        i32 = lax.iota(jnp.int32, 16) * _EMBED

        @pl.when(wid == 0)
        def _():
            pltpu.sync_copy(tail_hbm, tail_v)
            pltpu.sync_copy(
                tail_v, tlin_hbm.at[pl.ds((_NCOLT - 1) * 4096, 2048)])

        def col_of(kk):
            return wid + _NW * kk

        def in_desc(kk, b):
            col = col_of(kk)
            return pltpu.make_async_copy(
                tt_hbm.at[:, pl.ds(col * 128, 128)],
                src_v[b], isem[b])

        def out_desc(kk, b):
            col = col_of(kk)
            return pltpu.make_async_copy(
                dst_v[b], tlin_hbm.at[pl.ds(col * 4096, 4096)],
                osem[b])

        def start_in(kk, b):
            in_desc(kk, b).start()

        def wait_in(kk, b):
            in_desc(kk, b).wait()

        def start_out(kk, b):
            out_desc(kk, b).start()

        def wait_out(kk, b):
            out_desc(kk, b).wait()

        def work(kk, b):
            del kk
            _transpose_block(src_v[b], dst_v[b], 8, i32)

        def live(kk):
            return col_of(kk) < _NCOLT - 1

        @pl.when(live(0))
        def _():
            start_in(0, 0)

        def body2(g, carry):
            for half in range(2):
                kk = 2 * g + half
                b = half
                nb = 1 - half

                @pl.when(jnp.logical_and(kk >= 1, live(kk - 1)))
                def _():
                    wait_out(kk - 1, nb)

                @pl.when(live(kk + 1))
                def _():
                    start_in(kk + 1, nb)

                @pl.when(live(kk))
                def _():
                    wait_in(kk, b)
                    work(kk, b)
                    start_out(kk, b)
            return carry

        lax.fori_loop(0, (_KMAX - 1) // 2, body2, 0)

        @pl.when(live(_KMAX - 2))
        def _():
            wait_out(_KMAX - 2, 1)

        @pl.when(live(_KMAX - 1))
        def _():
            wait_in(_KMAX - 1, 0)
            work(_KMAX - 1, 0)
            start_out(_KMAX - 1, 0)
            wait_out(_KMAX - 1, 0)

    return k(table_t, tail_flat)


@jax.jit
def _embedding_sc(idx_flat, table):
    mesh = plsc.VectorSubcoreMesh(core_axis_name="c", subcore_axis_name="s")

    @functools.partial(
        pl.kernel,
        mesh=mesh,
        out_type=jax.ShapeDtypeStruct((_B, _EMBED), jnp.float32),
        scratch_types=[
            pltpu.VMEM((_BPW,), jnp.int32),
            pltpu.VMEM((2, _CHUNK, _EMBED), jnp.float32),
            pltpu.SemaphoreType.DMA((2,)),
            pltpu.SemaphoreType.DMA((2,)),
        ],
        compiler_params=pltpu.CompilerParams(use_tc_tiling_on_sc=False),
    )
    def k(idx_hbm, table_hbm, out_hbm, idx_v, rows_v, gsem, wsem):
        wid = lax.axis_index("s") * _NC + lax.axis_index("c")
        base = wid * _BPW
        pltpu.sync_copy(idx_hbm.at[pl.ds(base, _BPW)], idx_v)

        def g_desc(i, b):
            return pltpu.make_async_copy(
                table_hbm.at[idx_v.at[pl.ds(i * _CHUNK, _CHUNK)]],
                rows_v.at[b], gsem.at[b])

        def w_desc(i, b):
            return pltpu.make_async_copy(
                rows_v.at[b],
                out_hbm.at[pl.ds(base + i * _CHUNK, _CHUNK)], wsem.at[b])

        g_desc(0, 0).start()
        for i in range(_NCHUNK):
            b = i % 2
            nb = (i + 1) % 2
            if i + 1 < _NCHUNK:
                if i >= 1:
                    w_desc(i - 1, nb).wait()
                g_desc(i + 1, nb).start()
            g_desc(i, b).wait()
            w_desc(i, b).start()
        w_desc(_NCHUNK - 2, (_NCHUNK - 2) % 2).wait()
        w_desc(_NCHUNK - 1, (_NCHUNK - 1) % 2).wait()

    return k(idx_flat, table)


def kernel(inputs, table):
    idx_flat = inputs.reshape(-1).astype(jnp.int32)
    tail_flat = table[(_NCOLT - 1) * 128:, :].reshape(-1)
    tlin = _table_to_rowmajor(table.T, tail_flat)
    out = _embedding_sc(idx_flat, tlin.reshape(_VOCAB, _EMBED))
    return out.reshape(_BATCH, _MAX_LEN, _EMBED)


# speedup vs baseline: 1.2944x; 1.2944x over previous
import functools

import jax
import jax.numpy as jnp
from jax import lax
from jax.experimental import pallas as pl
from jax.experimental.pallas import tpu as pltpu
from jax.experimental.pallas import tpu_sc as plsc

_BATCH = 4096
_MAX_LEN = 200
_EMBED = 32
_VOCAB = 1000000
_B = _BATCH * _MAX_LEN
_NC = 2
_NS = 16
_NW = _NC * _NS

_BPW = _B // _NW
_CHUNK = 1600
_NCHUNK = _BPW // _CHUNK

_TBLK = 32
_TGRID = (_VOCAB + _TBLK * 128 - 1) // (_TBLK * 128)
_OUTR = _TBLK * 128 * _EMBED // 128
_VPAD = _TGRID * _TBLK * 128


@jax.jit
def _relayout_tc(table_t):
    def k(tt_ref, out_ref):
        x = tt_ref[...]
        y = x.T
        y3 = y.reshape(_OUTR, 4, _EMBED)
        out_ref[...] = jnp.concatenate(
            [y3[:, g, :] for g in range(4)], axis=1)

    return pl.pallas_call(
        k,
        grid=(_TGRID,),
        in_specs=[pl.BlockSpec((_EMBED, _TBLK * 128), lambda i: (0, i))],
        out_specs=pl.BlockSpec((_OUTR, 128), lambda i: (i, 0)),
        out_shape=jax.ShapeDtypeStruct((_TGRID * _OUTR, 128), jnp.float32),
    )(table_t)


@jax.jit
def _embedding_sc(idx_flat, table):
    mesh = plsc.VectorSubcoreMesh(core_axis_name="c", subcore_axis_name="s")

    @functools.partial(
        pl.kernel,
        mesh=mesh,
        out_type=jax.ShapeDtypeStruct((_B, _EMBED), jnp.float32),
        scratch_types=[
            pltpu.VMEM((_BPW,), jnp.int32),
            pltpu.VMEM((2, _CHUNK, _EMBED), jnp.float32),
            pltpu.SemaphoreType.DMA((2,)),
            pltpu.SemaphoreType.DMA((2,)),
        ],
        compiler_params=pltpu.CompilerParams(use_tc_tiling_on_sc=False),
    )
    def k(idx_hbm, table_hbm, out_hbm, idx_v, rows_v, gsem, wsem):
        wid = lax.axis_index("s") * _NC + lax.axis_index("c")
        base = wid * _BPW
        pltpu.sync_copy(idx_hbm.at[pl.ds(base, _BPW)], idx_v)

        def g_desc(i, b):
            return pltpu.make_async_copy(
                table_hbm.at[idx_v.at[pl.ds(i * _CHUNK, _CHUNK)]],
                rows_v.at[b], gsem.at[b])

        def w_desc(i, b):
            return pltpu.make_async_copy(
                rows_v.at[b],
                out_hbm.at[pl.ds(base + i * _CHUNK, _CHUNK)], wsem.at[b])

        g_desc(0, 0).start()
        for i in range(_NCHUNK):
            b = i % 2
            nb = (i + 1) % 2
            if i + 1 < _NCHUNK:
                if i >= 1:
                    w_desc(i - 1, nb).wait()
                g_desc(i + 1, nb).start()
            g_desc(i, b).wait()
            w_desc(i, b).start()
        w_desc(_NCHUNK - 2, (_NCHUNK - 2) % 2).wait()
        w_desc(_NCHUNK - 1, (_NCHUNK - 1) % 2).wait()

    return k(idx_flat, table)


def kernel(inputs, table):
    idx_flat = inputs.reshape(-1).astype(jnp.int32)
    tlin = _relayout_tc(table.T)
    out = _embedding_sc(idx_flat, tlin.reshape(_VPAD, _EMBED))
    return out.reshape(_BATCH, _MAX_LEN, _EMBED)
